# Initial kernel scaffold; baseline (speedup 1.0000x reference)
import jax
import jax.numpy as jnp
from jax.experimental import pallas as pl


def kernel(x, pos_x_w, pos_y_w, value_w, classify_w):
    raise NotImplementedError("write your pallas kernel here")


# speedup vs baseline: 26.1210x; 26.1210x over previous
import jax
import jax.numpy as jnp
from jax.experimental import pallas as pl
from jax.experimental.pallas import tpu as pltpu

NUM_LEVELS = 10


def _tc_body(x_ref, posx_ref, posy_ref, val_ref, cls_ref, out_ref):
    x = x_ref[:]
    B = x.shape[0]
    size = x.shape[1]
    L = NUM_LEVELS
    r = jnp.clip(jnp.round(x * (L - 1)), 0.0, float(L - 1))
    cx_list = []
    cy_list = []
    for l in range(L):
        m = (r == float(l)).astype(jnp.float32)
        cx_list.append(jnp.sum(m, axis=1))
        cy_list.append(jnp.sum(m, axis=2))
    cx = jnp.stack(cx_list, axis=1)
    cy = jnp.stack(cy_list, axis=1)
    ax = jnp.dot(cx.reshape(B * L, size), posx_ref[:],
                 preferred_element_type=jnp.float32)
    ay = jnp.dot(cy.reshape(B * L, size), posy_ref[:],
                 preferred_element_type=jnp.float32)
    a = (ax + ay).reshape(B, L, -1)
    multiset = jnp.sum(a * val_ref[:][None, :, :], axis=1)
    enc = jnp.where(multiset > 0, 1.0, -1.0)
    out_ref[:] = jnp.dot(enc, cls_ref[:].T, preferred_element_type=jnp.float32)


def kernel(x, pos_x_w, pos_y_w, value_w, classify_w):
    B = x.shape[0]
    num_classes = classify_w.shape[0]
    return pl.pallas_call(
        _tc_body,
        out_shape=jax.ShapeDtypeStruct((B, num_classes), jnp.float32),
    )(x, pos_x_w, pos_y_w, value_w, classify_w)
